# baseline (speedup 1.0000x reference)
import jax
import jax.numpy as jnp
from jax import lax
from jax.experimental import pallas as pl
from jax.experimental.pallas import tpu as pltpu
from jax.experimental.pallas import tpu_sc as plsc

NC = 2
NS = 16
NW = NC * NS

EMB_DIM = 32
CHUNK = 128
K = 8
GROUP = K * CHUNK

TBLK = 1024


def _transpose_body(tT_ref, lin_ref):
    x = tT_ref[...]
    q = TBLK // 4
    lin_ref[...] = jnp.concatenate(
        [x[:, i * q:(i + 1) * q].T for i in range(4)], axis=1)


def _gather_body(idxT_hbm, table_hbm, out_hbm, idx_v, rows0, rows1,
                 g0, g1, s0, s1):
    nwalk, nbatch = idxT_hbm.shape
    bp = nbatch // NW
    cpw = bp // CHUNK
    ngroups = (nwalk * cpw) // K
    t = lax.axis_index("s") * NC + lax.axis_index("c")
    b0 = t * bp

    pltpu.sync_copy(idxT_hbm.at[:, pl.ds(b0, bp)], idx_v)

    bufs = (rows0, rows1)
    gsems = (g0, g1)
    ssems = (s0, s1)
    wpg = K // cpw

    def fire_gathers(grp, b):
        for i in range(K):
            w = grp * wpg + i // cpw
            c = i % cpw
            pltpu.async_copy(
                table_hbm.at[idx_v.at[w, pl.ds(c * CHUNK, CHUNK)]],
                bufs[b].at[pl.ds(i * CHUNK, CHUNK)],
                gsems[b])

    def drain_gathers(b):
        pltpu.make_async_copy(
            out_hbm.at[0, pl.ds(0, GROUP)], bufs[b], gsems[b]).wait()

    def fire_stores(grp, b):
        for i in range(K):
            w = grp * wpg + i // cpw
            c = i % cpw
            pltpu.async_copy(
                bufs[b].at[pl.ds(i * CHUNK, CHUNK)],
                out_hbm.at[w, pl.ds(b0 + c * CHUNK, CHUNK)],
                ssems[b])

    def drain_stores(b):
        pltpu.make_async_copy(
            bufs[b], out_hbm.at[0, pl.ds(0, GROUP)], ssems[b]).wait()

    fire_gathers(0, 0)
    fire_gathers(1, 1)

    @pl.loop(0, ngroups - 2, step=2)
    def _(g):
        drain_gathers(0)
        fire_stores(g, 0)
        drain_gathers(1)
        fire_stores(g + 1, 1)
        drain_stores(0)
        fire_gathers(g + 2, 0)
        drain_stores(1)
        fire_gathers(g + 3, 1)

    drain_gathers(0)
    fire_stores(ngroups - 2, 0)
    drain_gathers(1)
    fire_stores(ngroups - 1, 1)
    drain_stores(0)
    drain_stores(1)


def kernel(indices, emb_table):
    b, w = indices.shape
    n_nodes, d = emb_table.shape
    idxT = indices.astype(jnp.int32).T
    tT = emb_table.T

    idxT = ((idxT >> 10) << 10) | ((idxT & 255) << 2) | ((idxT & 1023) >> 8)

    nblk = pl.cdiv(n_nodes, TBLK)
    rblk = TBLK // 4
    lin = pl.pallas_call(
        _transpose_body,
        grid=(nblk,),
        in_specs=[pl.BlockSpec((d, TBLK), lambda g: (0, g))],
        out_specs=pl.BlockSpec((rblk, 128), lambda g: (g, 0)),
        out_shape=jax.ShapeDtypeStruct((nblk * rblk, 128), jnp.float32),
    )(tT)
    table_lin = lin.reshape(nblk * TBLK, d)

    run = pl.kernel(
        _gather_body,
        out_type=jax.ShapeDtypeStruct((w, b, EMB_DIM), jnp.float32),
        mesh=plsc.VectorSubcoreMesh(
            core_axis_name="c", subcore_axis_name="s",
            num_cores=NC, num_subcores=NS),
        scratch_types=[
            pltpu.VMEM((w, b // NW), jnp.int32),
            pltpu.VMEM((GROUP, EMB_DIM), jnp.float32),
            pltpu.VMEM((GROUP, EMB_DIM), jnp.float32),
            pltpu.SemaphoreType.DMA,
            pltpu.SemaphoreType.DMA,
            pltpu.SemaphoreType.DMA,
            pltpu.SemaphoreType.DMA,
        ],
        compiler_params=pltpu.CompilerParams(use_tc_tiling_on_sc=False),
    )
    out = run(idxT, table_lin)
    return out.transpose(1, 0, 2)
